# baseline (speedup 1.0000x reference)
import jax
import jax.numpy as jnp
from jax.experimental import pallas as pl
from jax.experimental.pallas import tpu as pltpu

_N_CHUNKS = 4


def _copy_body(in_hbm, out_hbm, sems):
    rows = in_hbm.shape[0]
    chunk = rows // _N_CHUNKS
    copies = [
        pltpu.make_async_copy(
            in_hbm.at[pl.ds(i * chunk, chunk), :],
            out_hbm.at[pl.ds(i * chunk, chunk), :],
            sems.at[i],
        )
        for i in range(_N_CHUNKS)
    ]
    for c in copies:
        c.start()
    for c in copies:
        c.wait()


def kernel(x, embed_weight):
    del embed_weight
    b, s, d = x.shape
    rows = b * s
    x2 = x.reshape(rows, d)
    out = pl.pallas_call(
        _copy_body,
        out_shape=jax.ShapeDtypeStruct((rows, d), x.dtype),
        in_specs=[pl.BlockSpec(memory_space=pltpu.MemorySpace.HBM)],
        out_specs=pl.BlockSpec(memory_space=pltpu.MemorySpace.HBM),
        scratch_shapes=[pltpu.SemaphoreType.DMA((_N_CHUNKS,))],
    )(x2)
    return out.reshape(b, s, d)


# speedup vs baseline: 45.6442x; 45.6442x over previous
import jax
import jax.numpy as jnp
from jax.experimental import pallas as pl
from jax.experimental.pallas import tpu as pltpu


def _copy_body(in_ref, out_ref):
    out_ref[...] = in_ref[...]


def kernel(x, embed_weight):
    del embed_weight
    b, s, d = x.shape
    rows = b * s
    x2 = x.reshape(rows, d)
    block_rows = 1024
    out = pl.pallas_call(
        _copy_body,
        out_shape=jax.ShapeDtypeStruct((rows, d), x.dtype),
        grid=(rows // block_rows,),
        in_specs=[pl.BlockSpec((block_rows, d), lambda i: (i, 0))],
        out_specs=pl.BlockSpec((block_rows, d), lambda i: (i, 0)),
        compiler_params=pltpu.CompilerParams(
            dimension_semantics=("parallel",),
        ),
    )(x2)
    return out.reshape(b, s, d)


# speedup vs baseline: 48.8768x; 1.0708x over previous
import jax
import jax.numpy as jnp
from jax.experimental import pallas as pl
from jax.experimental.pallas import tpu as pltpu


def _copy_body(in_ref, out_ref):
    out_ref[...] = in_ref[...]


def kernel(x, embed_weight):
    del embed_weight
    b, s, d = x.shape
    rows = b * s
    x2 = x.reshape(rows, d)
    block_rows = 2048
    out = pl.pallas_call(
        _copy_body,
        out_shape=jax.ShapeDtypeStruct((rows, d), x.dtype),
        grid=(rows // block_rows,),
        in_specs=[pl.BlockSpec((block_rows, d), lambda i: (i, 0))],
        out_specs=pl.BlockSpec((block_rows, d), lambda i: (i, 0)),
        compiler_params=pltpu.CompilerParams(
            dimension_semantics=("parallel",),
        ),
    )(x2)
    return out.reshape(b, s, d)
